# baseline (speedup 1.0000x reference)
import functools

import jax
import jax.numpy as jnp
from jax import lax
from jax.experimental import pallas as pl
from jax.experimental.pallas import tpu as pltpu
from jax.experimental.pallas import tpu_sc as plsc

_THRESH = 0.7
_MIN_KEPT = 100000
_C = 19
_B = 8
_NC, _NS, _L = 2, 16, 16
_NW = _NC * _NS
_W = 512
_CR = 4
_CH = _CR * _W
_PW = _B * 512 * _W // _NW
_RW = _PW // _W
_QW = 512 // _RW
_NCHUNK = _RW // _CR
_NG = _CH // _L

_LN2 = 0.6931471805599453


def _vlog(s):
    bits = lax.bitcast_convert_type(s, jnp.int32)
    e = (bits >> 23) - 127
    mant = lax.bitcast_convert_type((bits & 0x7FFFFF) | 0x3F800000, jnp.float32)
    u = (mant - 1.0) / (mant + 1.0)
    u2 = u * u
    p = 1.0 + u2 * ((1.0 / 3.0) + u2 * ((1.0 / 5.0) + u2 * ((1.0 / 7.0) + u2 * (1.0 / 9.0))))
    return e.astype(jnp.float32) * _LN2 + 2.0 * u * p


def _sc_body(score_hbm, tgt_hbm, thr_hbm, out_hbm,
             buf_a, buf_b, tbuf_a, tbuf_b, thrv, accv,
             sem_a, sem_b, tsem_a, tsem_b):
    wid = lax.axis_index("s") * _NC + lax.axis_index("c")
    b = wid // _QW
    base_row = (wid % _QW) * _RW

    pltpu.sync_copy(thr_hbm, thrv)
    thr = thrv[...]

    def start(j, buf, tbuf, sem, tsem):
        row = base_row + j * _CR
        for c in range(_C):
            pltpu.make_async_copy(score_hbm.at[b, c, pl.ds(row, _CR), :],
                                  buf.at[pl.ds(c * _CR, _CR), :], sem).start()
        pltpu.make_async_copy(tgt_hbm.at[b, pl.ds(row, _CR), :], tbuf, tsem).start()

    def wait(j, buf, tbuf, sem, tsem):
        row = base_row + j * _CR
        for c in range(_C):
            pltpu.make_async_copy(score_hbm.at[b, c, pl.ds(row, _CR), :],
                                  buf.at[pl.ds(c * _CR, _CR), :], sem).wait()
        pltpu.make_async_copy(tgt_hbm.at[b, pl.ds(row, _CR), :], tbuf, tsem).wait()

    start(0, buf_a, tbuf_a, sem_a, tsem_a)
    start(1, buf_b, tbuf_b, sem_b, tsem_b)

    one = jnp.ones((_L,), jnp.float32)
    zero = jnp.zeros((_L,), jnp.float32)

    def tree(op, vals):
        vals = list(vals)
        while len(vals) > 1:
            nxt = [op(vals[i], vals[i + 1]) for i in range(0, len(vals) - 1, 2)]
            if len(vals) % 2:
                nxt.append(vals[-1])
            vals = nxt
        return vals[0]

    def chunk(j, buf, tbuf, sem, tsem, accs):
        wait(j, buf, tbuf, sem, tsem)

        def one_group(r, wb, accs):
            c_lt, s_lt, c_le = accs
            xs = [buf[c * _CR + r, pl.ds(wb, _L)] for c in range(_C)]
            tv = tbuf[r, pl.ds(wb, _L)]
            x_t = tree(jnp.add,
                       [jnp.where(tv == c, xs[c], zero) for c in range(_C)])
            m = tree(jnp.maximum, xs)
            ssum = tree(jnp.add, [jnp.exp(x - m) for x in xs])
            e_t = jnp.exp(x_t - m)
            ts = thr * ssum
            nll = _vlog(ssum) + (m - x_t)
            lt = e_t < ts
            return (c_lt + jnp.where(lt, one, zero),
                    s_lt + jnp.where(lt, nll, zero),
                    c_le + jnp.where(e_t <= ts, one, zero))

        def grp(g, accs):
            g2 = g * 2
            r = g2 >> 5
            wb = (g2 & 31) * _L
            accs = one_group(r, wb, accs)
            return one_group(r, wb + _L, accs)

        accs = lax.fori_loop(0, _NG // 2, grp, accs)

        @pl.when(j + 2 < _NCHUNK)
        def _():
            start(j + 2, buf, tbuf, sem, tsem)

        return accs

    zero16 = jnp.zeros((_L,), jnp.float32)

    def pair(i, accs):
        accs = chunk(2 * i, buf_a, tbuf_a, sem_a, tsem_a, accs)
        accs = chunk(2 * i + 1, buf_b, tbuf_b, sem_b, tsem_b, accs)
        return accs

    accs = lax.fori_loop(0, _NCHUNK // 2, pair, (zero16, zero16, zero16))

    accv[pl.ds(0, _L)] = accs[0]
    accv[pl.ds(_L, _L)] = accs[1]
    accv[pl.ds(2 * _L, _L)] = accs[2]
    pltpu.sync_copy(accv, out_hbm.at[wid])


_sc_stats_kernel = functools.partial(
    pl.kernel,
    out_type=jax.ShapeDtypeStruct((_NW, 3 * _L), jnp.float32),
    mesh=plsc.VectorSubcoreMesh(core_axis_name="c", subcore_axis_name="s",
                                num_cores=_NC, num_subcores=_NS),
    compiler_params=pltpu.CompilerParams(use_tc_tiling_on_sc=True),
    scratch_types=[
        pltpu.VMEM((_C * _CR, _W), jnp.float32),
        pltpu.VMEM((_C * _CR, _W), jnp.float32),
        pltpu.VMEM((_CR, _W), jnp.int32),
        pltpu.VMEM((_CR, _W), jnp.int32),
        pltpu.VMEM((_L,), jnp.float32),
        pltpu.VMEM((3 * _L,), jnp.float32),
        pltpu.SemaphoreType.DMA,
        pltpu.SemaphoreType.DMA,
        pltpu.SemaphoreType.DMA,
        pltpu.SemaphoreType.DMA,
    ],
)(_sc_body)


def _sc_stats(score, target, thr):
    thrv = jnp.full((_L,), thr, jnp.float32)
    out = _sc_stats_kernel(score, target, thrv)
    return (jnp.sum(out[:, 0:_L]),
            jnp.sum(out[:, _L:2 * _L]),
            jnp.sum(out[:, 2 * _L:3 * _L]))


def kernel(score, target):
    kp1 = jnp.float32(_MIN_KEPT + 1)
    cnt_lt, sum_lt, cnt_le = _sc_stats(score, target, jnp.float32(_THRESH))

    def case_a(_):
        return sum_lt / jnp.maximum(cnt_lt, 1.0)

    def case_b(_):
        def cond(st):
            lo, hi = st
            return hi - lo > 1

        def body(st):
            lo, hi = st
            mid = (lo + hi) // 2
            t = lax.bitcast_convert_type(mid, jnp.float32)
            _, _, c_le = _sc_stats(score, target, t)
            ge = c_le >= kp1
            return jnp.where(ge, lo, mid), jnp.where(ge, mid, hi)

        lo0 = jnp.int32(0x3F333333)
        hi0 = jnp.int32(0x3F800000)
        _, hi = lax.while_loop(cond, body, (lo0, hi0))
        vstar = lax.bitcast_convert_type(hi, jnp.float32)
        c_lt2, s_lt2, _ = _sc_stats(score, target, vstar)
        return s_lt2 / jnp.maximum(c_lt2, 1.0)

    return lax.cond(cnt_le < kp1, case_b, case_a, None)


# speedup vs baseline: 2.7532x; 2.7532x over previous
import functools

import jax
import jax.numpy as jnp
from jax import lax
from jax.experimental import pallas as pl
from jax.experimental.pallas import tpu as pltpu
from jax.experimental.pallas import tpu_sc as plsc

_THRESH = 0.7
_MIN_KEPT = 100000
_C = 19
_B = 8
_B_TC = 6
_B0 = _B_TC
_BH = 64

_NC, _NS, _L = 2, 16, 16
_NW = _NC * _NS
_W = 512
_CR = 4
_PW = (_B - _B0) * 512 * _W // _NW
_RW = _PW // _W
_QW = 512 // _RW
_NCHUNK = _RW // _CR
_NG = _CR * _W // _L

_LN2 = 0.6931471805599453


def _tc_body(thr_ref, score_ref, tgt_ref, cnt_lt_ref, sum_lt_ref, cnt_le_ref):
    i = pl.program_id(0)
    j = pl.program_id(1)

    @pl.when((i == 0) & (j == 0))
    def _init():
        cnt_lt_ref[0, 0] = 0.0
        sum_lt_ref[0, 0] = 0.0
        cnt_le_ref[0, 0] = 0.0

    x = score_ref[0]
    t = tgt_ref[0]
    m = jnp.max(x, axis=0)
    onehot = lax.broadcasted_iota(jnp.int32, x.shape, 0) == t[None]
    x_t = jnp.sum(jnp.where(onehot, x, 0.0), axis=0)
    e = jnp.exp(x - m[None])
    s = jnp.sum(e, axis=0)
    e_t = jnp.exp(x_t - m)
    pg = e_t / s
    nll = jnp.log(s) + (m - x_t)
    thr = thr_ref[0, 0]
    lt = pg < thr
    cnt_lt_ref[0, 0] += jnp.sum(lt.astype(jnp.float32))
    sum_lt_ref[0, 0] += jnp.sum(jnp.where(lt, nll, 0.0))
    cnt_le_ref[0, 0] += jnp.sum((pg <= thr).astype(jnp.float32))


def _tc_stats(score, target, thrv2):
    out = pl.pallas_call(
        _tc_body,
        grid=(_B_TC, 512 // _BH),
        in_specs=[
            pl.BlockSpec(memory_space=pltpu.SMEM),
            pl.BlockSpec((1, _C, _BH, _W), lambda i, j: (i, 0, j, 0)),
            pl.BlockSpec((1, _BH, _W), lambda i, j: (i, j, 0)),
        ],
        out_specs=[
            pl.BlockSpec(memory_space=pltpu.SMEM),
            pl.BlockSpec(memory_space=pltpu.SMEM),
            pl.BlockSpec(memory_space=pltpu.SMEM),
        ],
        out_shape=[jax.ShapeDtypeStruct((1, 1), jnp.float32)] * 3,
    )(thrv2, score, target)
    return out[0][0, 0], out[1][0, 0], out[2][0, 0]


def _vlog(s):
    bits = lax.bitcast_convert_type(s, jnp.int32)
    e = (bits >> 23) - 127
    mant = lax.bitcast_convert_type((bits & 0x7FFFFF) | 0x3F800000, jnp.float32)
    u = (mant - 1.0) / (mant + 1.0)
    u2 = u * u
    p = 1.0 + u2 * ((1.0 / 3.0) + u2 * ((1.0 / 5.0) + u2 * ((1.0 / 7.0) + u2 * (1.0 / 9.0))))
    return e.astype(jnp.float32) * _LN2 + 2.0 * u * p


def _sc_body(score_hbm, tgt_hbm, thr_hbm, out_hbm,
             buf_a, buf_b, tbuf_a, tbuf_b, thrv, accv,
             sem_a, sem_b, tsem_a, tsem_b):
    wid = lax.axis_index("s") * _NC + lax.axis_index("c")
    b = _B0 + wid // _QW
    base_row = (wid % _QW) * _RW

    pltpu.sync_copy(thr_hbm, thrv)
    thr = thrv[...]

    def start(j, buf, tbuf, sem, tsem):
        row = base_row + j * _CR
        for c in range(_C):
            pltpu.make_async_copy(score_hbm.at[b, c, pl.ds(row, _CR), :],
                                  buf.at[pl.ds(c * _CR, _CR), :], sem).start()
        pltpu.make_async_copy(tgt_hbm.at[b, pl.ds(row, _CR), :], tbuf, tsem).start()

    def wait(j, buf, tbuf, sem, tsem):
        row = base_row + j * _CR
        for c in range(_C):
            pltpu.make_async_copy(score_hbm.at[b, c, pl.ds(row, _CR), :],
                                  buf.at[pl.ds(c * _CR, _CR), :], sem).wait()
        pltpu.make_async_copy(tgt_hbm.at[b, pl.ds(row, _CR), :], tbuf, tsem).wait()

    start(0, buf_a, tbuf_a, sem_a, tsem_a)
    start(1, buf_b, tbuf_b, sem_b, tsem_b)

    one = jnp.ones((_L,), jnp.float32)
    zero = jnp.zeros((_L,), jnp.float32)

    def tree(op, vals):
        vals = list(vals)
        while len(vals) > 1:
            nxt = [op(vals[i], vals[i + 1]) for i in range(0, len(vals) - 1, 2)]
            if len(vals) % 2:
                nxt.append(vals[-1])
            vals = nxt
        return vals[0]

    def chunk(j, buf, tbuf, sem, tsem, accs):
        wait(j, buf, tbuf, sem, tsem)

        def one_group(r, wb, accs):
            c_lt, s_lt, c_le = accs
            xs = [buf[c * _CR + r, pl.ds(wb, _L)] for c in range(_C)]
            tv = tbuf[r, pl.ds(wb, _L)]
            x_t = tree(jnp.add,
                       [jnp.where(tv == c, xs[c], zero) for c in range(_C)])
            m = tree(jnp.maximum, xs)
            ssum = tree(jnp.add, [jnp.exp(x - m) for x in xs])
            e_t = jnp.exp(x_t - m)
            ts = thr * ssum
            nll = _vlog(ssum) + (m - x_t)
            lt = e_t < ts
            return (c_lt + jnp.where(lt, one, zero),
                    s_lt + jnp.where(lt, nll, zero),
                    c_le + jnp.where(e_t <= ts, one, zero))

        def grp(g, accs):
            g2 = g * 2
            r = g2 >> 5
            wb = (g2 & 31) * _L
            accs = one_group(r, wb, accs)
            return one_group(r, wb + _L, accs)

        accs = lax.fori_loop(0, _NG // 2, grp, accs)

        @pl.when(j + 2 < _NCHUNK)
        def _():
            start(j + 2, buf, tbuf, sem, tsem)

        return accs

    zero16 = jnp.zeros((_L,), jnp.float32)

    def pair(i, accs):
        accs = chunk(2 * i, buf_a, tbuf_a, sem_a, tsem_a, accs)
        accs = chunk(2 * i + 1, buf_b, tbuf_b, sem_b, tsem_b, accs)
        return accs

    accs = lax.fori_loop(0, _NCHUNK // 2, pair, (zero16, zero16, zero16))

    accv[pl.ds(0, _L)] = accs[0]
    accv[pl.ds(_L, _L)] = accs[1]
    accv[pl.ds(2 * _L, _L)] = accs[2]
    pltpu.sync_copy(accv, out_hbm.at[wid])


_sc_stats_kernel = functools.partial(
    pl.kernel,
    out_type=jax.ShapeDtypeStruct((_NW, 3 * _L), jnp.float32),
    mesh=plsc.VectorSubcoreMesh(core_axis_name="c", subcore_axis_name="s",
                                num_cores=_NC, num_subcores=_NS),
    compiler_params=pltpu.CompilerParams(use_tc_tiling_on_sc=True),
    scratch_types=[
        pltpu.VMEM((_C * _CR, _W), jnp.float32),
        pltpu.VMEM((_C * _CR, _W), jnp.float32),
        pltpu.VMEM((_CR, _W), jnp.int32),
        pltpu.VMEM((_CR, _W), jnp.int32),
        pltpu.VMEM((_L,), jnp.float32),
        pltpu.VMEM((3 * _L,), jnp.float32),
        pltpu.SemaphoreType.DMA,
        pltpu.SemaphoreType.DMA,
        pltpu.SemaphoreType.DMA,
        pltpu.SemaphoreType.DMA,
    ],
)(_sc_body)


def _stats(score, target, thr):
    thr = jnp.asarray(thr, jnp.float32)
    sc_out = _sc_stats_kernel(score, target, jnp.full((_L,), thr, jnp.float32))
    tc_lt, tc_sum, tc_le = _tc_stats(score, target, thr.reshape(1, 1))
    return (tc_lt + jnp.sum(sc_out[:, 0:_L]),
            tc_sum + jnp.sum(sc_out[:, _L:2 * _L]),
            tc_le + jnp.sum(sc_out[:, 2 * _L:3 * _L]))


def kernel(score, target):
    kp1 = jnp.float32(_MIN_KEPT + 1)
    cnt_lt, sum_lt, cnt_le = _stats(score, target, _THRESH)

    def case_a(_):
        return sum_lt / jnp.maximum(cnt_lt, 1.0)

    def case_b(_):
        def cond(st):
            lo, hi = st
            return hi - lo > 1

        def body(st):
            lo, hi = st
            mid = (lo + hi) // 2
            t = lax.bitcast_convert_type(mid, jnp.float32)
            _, _, c_le = _stats(score, target, t)
            ge = c_le >= kp1
            return jnp.where(ge, lo, mid), jnp.where(ge, mid, hi)

        lo0 = jnp.int32(0x3F333333)
        hi0 = jnp.int32(0x3F800000)
        _, hi = lax.while_loop(cond, body, (lo0, hi0))
        vstar = lax.bitcast_convert_type(hi, jnp.float32)
        c_lt2, s_lt2, _ = _stats(score, target, vstar)
        return s_lt2 / jnp.maximum(c_lt2, 1.0)

    return lax.cond(cnt_le < kp1, case_b, case_a, None)


# speedup vs baseline: 2.8035x; 1.0183x over previous
import functools

import jax
import jax.numpy as jnp
from jax import lax
from jax.experimental import pallas as pl
from jax.experimental.pallas import tpu as pltpu
from jax.experimental.pallas import tpu_sc as plsc

_THRESH = 0.7
_MIN_KEPT = 100000
_C = 19
_B = 8
_B_TC = 6
_B0 = _B_TC
_BH = 128

_NC, _NS, _L = 2, 16, 16
_NW = _NC * _NS
_W = 512
_CR = 4
_PW = (_B - _B0) * 512 * _W // _NW
_RW = _PW // _W
_QW = 512 // _RW
_NCHUNK = _RW // _CR
_NG = _CR * _W // _L

_LN2 = 0.6931471805599453


def _tc_body(thr_ref, score_ref, tgt_ref, cnt_lt_ref, sum_lt_ref, cnt_le_ref):
    i = pl.program_id(0)
    j = pl.program_id(1)

    @pl.when((i == 0) & (j == 0))
    def _init():
        cnt_lt_ref[0, 0] = 0.0
        sum_lt_ref[0, 0] = 0.0
        cnt_le_ref[0, 0] = 0.0

    x = score_ref[0]
    t = tgt_ref[0]
    m = jnp.max(x, axis=0)
    onehot = lax.broadcasted_iota(jnp.int32, x.shape, 0) == t[None]
    x_t = jnp.sum(jnp.where(onehot, x, 0.0), axis=0)
    e = jnp.exp(x - m[None])
    s = jnp.sum(e, axis=0)
    e_t = jnp.exp(x_t - m)
    nll = jnp.log(s) + (m - x_t)
    ts = thr_ref[0, 0] * s
    lt = e_t < ts
    cnt_lt_ref[0, 0] += jnp.sum(lt.astype(jnp.float32))
    sum_lt_ref[0, 0] += jnp.sum(jnp.where(lt, nll, 0.0))
    cnt_le_ref[0, 0] += jnp.sum((e_t <= ts).astype(jnp.float32))


def _tc_stats(score, target, thrv2):
    out = pl.pallas_call(
        _tc_body,
        grid=(_B_TC, 512 // _BH),
        in_specs=[
            pl.BlockSpec(memory_space=pltpu.SMEM),
            pl.BlockSpec((1, _C, _BH, _W), lambda i, j: (i, 0, j, 0)),
            pl.BlockSpec((1, _BH, _W), lambda i, j: (i, j, 0)),
        ],
        out_specs=[
            pl.BlockSpec(memory_space=pltpu.SMEM),
            pl.BlockSpec(memory_space=pltpu.SMEM),
            pl.BlockSpec(memory_space=pltpu.SMEM),
        ],
        out_shape=[jax.ShapeDtypeStruct((1, 1), jnp.float32)] * 3,
    )(thrv2, score, target)
    return out[0][0, 0], out[1][0, 0], out[2][0, 0]


def _vlog(s):
    bits = lax.bitcast_convert_type(s, jnp.int32)
    e = (bits >> 23) - 127
    mant = lax.bitcast_convert_type((bits & 0x7FFFFF) | 0x3F800000, jnp.float32)
    u = (mant - 1.0) / (mant + 1.0)
    u2 = u * u
    p = 1.0 + u2 * ((1.0 / 3.0) + u2 * ((1.0 / 5.0) + u2 * ((1.0 / 7.0) + u2 * (1.0 / 9.0))))
    return e.astype(jnp.float32) * _LN2 + 2.0 * u * p


def _sc_body(score_hbm, tgt_hbm, thr_hbm, out_hbm,
             buf_a, buf_b, tbuf_a, tbuf_b, thrv, accv,
             sem_a, sem_b, tsem_a, tsem_b):
    wid = lax.axis_index("s") * _NC + lax.axis_index("c")
    b = _B0 + wid // _QW
    base_row = (wid % _QW) * _RW

    pltpu.sync_copy(thr_hbm, thrv)
    thr = thrv[...]

    def start(j, buf, tbuf, sem, tsem):
        row = base_row + j * _CR
        for c in range(_C):
            pltpu.make_async_copy(score_hbm.at[b, c, pl.ds(row, _CR), :],
                                  buf.at[pl.ds(c * _CR, _CR), :], sem).start()
        pltpu.make_async_copy(tgt_hbm.at[b, pl.ds(row, _CR), :], tbuf, tsem).start()

    def wait(j, buf, tbuf, sem, tsem):
        row = base_row + j * _CR
        for c in range(_C):
            pltpu.make_async_copy(score_hbm.at[b, c, pl.ds(row, _CR), :],
                                  buf.at[pl.ds(c * _CR, _CR), :], sem).wait()
        pltpu.make_async_copy(tgt_hbm.at[b, pl.ds(row, _CR), :], tbuf, tsem).wait()

    start(0, buf_a, tbuf_a, sem_a, tsem_a)
    start(1, buf_b, tbuf_b, sem_b, tsem_b)

    one = jnp.ones((_L,), jnp.float32)
    zero = jnp.zeros((_L,), jnp.float32)

    def tree(op, vals):
        vals = list(vals)
        while len(vals) > 1:
            nxt = [op(vals[i], vals[i + 1]) for i in range(0, len(vals) - 1, 2)]
            if len(vals) % 2:
                nxt.append(vals[-1])
            vals = nxt
        return vals[0]

    def chunk(j, buf, tbuf, sem, tsem, accs):
        wait(j, buf, tbuf, sem, tsem)

        def one_group(r, wb, accs):
            c_lt, s_lt, c_le = accs
            xs = [buf[c * _CR + r, pl.ds(wb, _L)] for c in range(_C)]
            tv = tbuf[r, pl.ds(wb, _L)]
            x_t = tree(jnp.add,
                       [jnp.where(tv == c, xs[c], zero) for c in range(_C)])
            m = tree(jnp.maximum, xs)
            ssum = tree(jnp.add, [jnp.exp(x - m) for x in xs])
            e_t = jnp.exp(x_t - m)
            ts = thr * ssum
            nll = _vlog(ssum) + (m - x_t)
            lt = e_t < ts
            return (c_lt + jnp.where(lt, one, zero),
                    s_lt + jnp.where(lt, nll, zero),
                    c_le + jnp.where(e_t <= ts, one, zero))

        def grp(g, accs):
            g2 = g * 2
            r = g2 >> 5
            wb = (g2 & 31) * _L
            accs = one_group(r, wb, accs)
            return one_group(r, wb + _L, accs)

        accs = lax.fori_loop(0, _NG // 2, grp, accs)

        @pl.when(j + 2 < _NCHUNK)
        def _():
            start(j + 2, buf, tbuf, sem, tsem)

        return accs

    zero16 = jnp.zeros((_L,), jnp.float32)

    def pair(i, accs):
        accs = chunk(2 * i, buf_a, tbuf_a, sem_a, tsem_a, accs)
        accs = chunk(2 * i + 1, buf_b, tbuf_b, sem_b, tsem_b, accs)
        return accs

    accs = lax.fori_loop(0, _NCHUNK // 2, pair, (zero16, zero16, zero16))

    accv[pl.ds(0, _L)] = accs[0]
    accv[pl.ds(_L, _L)] = accs[1]
    accv[pl.ds(2 * _L, _L)] = accs[2]
    pltpu.sync_copy(accv, out_hbm.at[wid])


_sc_stats_kernel = functools.partial(
    pl.kernel,
    out_type=jax.ShapeDtypeStruct((_NW, 3 * _L), jnp.float32),
    mesh=plsc.VectorSubcoreMesh(core_axis_name="c", subcore_axis_name="s",
                                num_cores=_NC, num_subcores=_NS),
    compiler_params=pltpu.CompilerParams(use_tc_tiling_on_sc=True),
    scratch_types=[
        pltpu.VMEM((_C * _CR, _W), jnp.float32),
        pltpu.VMEM((_C * _CR, _W), jnp.float32),
        pltpu.VMEM((_CR, _W), jnp.int32),
        pltpu.VMEM((_CR, _W), jnp.int32),
        pltpu.VMEM((_L,), jnp.float32),
        pltpu.VMEM((3 * _L,), jnp.float32),
        pltpu.SemaphoreType.DMA,
        pltpu.SemaphoreType.DMA,
        pltpu.SemaphoreType.DMA,
        pltpu.SemaphoreType.DMA,
    ],
)(_sc_body)


def _stats(score, target, thr):
    thr = jnp.asarray(thr, jnp.float32)
    sc_out = _sc_stats_kernel(score, target, jnp.full((_L,), thr, jnp.float32))
    tc_lt, tc_sum, tc_le = _tc_stats(score, target, thr.reshape(1, 1))
    return (tc_lt + jnp.sum(sc_out[:, 0:_L]),
            tc_sum + jnp.sum(sc_out[:, _L:2 * _L]),
            tc_le + jnp.sum(sc_out[:, 2 * _L:3 * _L]))


def kernel(score, target):
    kp1 = jnp.float32(_MIN_KEPT + 1)
    cnt_lt, sum_lt, cnt_le = _stats(score, target, _THRESH)

    def case_a(_):
        return sum_lt / jnp.maximum(cnt_lt, 1.0)

    def case_b(_):
        def cond(st):
            lo, hi = st
            return hi - lo > 1

        def body(st):
            lo, hi = st
            mid = (lo + hi) // 2
            t = lax.bitcast_convert_type(mid, jnp.float32)
            _, _, c_le = _stats(score, target, t)
            ge = c_le >= kp1
            return jnp.where(ge, lo, mid), jnp.where(ge, mid, hi)

        lo0 = jnp.int32(0x3F333333)
        hi0 = jnp.int32(0x3F800000)
        _, hi = lax.while_loop(cond, body, (lo0, hi0))
        vstar = lax.bitcast_convert_type(hi, jnp.float32)
        c_lt2, s_lt2, _ = _stats(score, target, vstar)
        return s_lt2 / jnp.maximum(c_lt2, 1.0)

    return lax.cond(cnt_le < kp1, case_b, case_a, None)


# speedup vs baseline: 2.9761x; 1.0616x over previous
import functools

import jax
import jax.numpy as jnp
from jax import lax
from jax.experimental import pallas as pl
from jax.experimental.pallas import tpu as pltpu
from jax.experimental.pallas import tpu_sc as plsc

_THRESH = 0.7
_MIN_KEPT = 100000
_C = 19
_B = 8
_B_TC = 6
_B0 = _B_TC
_BH = 128

_NC, _NS, _L = 2, 16, 16
_NW = _NC * _NS
_W = 512
_CR = 4
_PW = (_B - _B0) * 512 * _W // _NW
_RW = _PW // _W
_QW = 512 // _RW
_NCHUNK = _RW // _CR
_NG = _CR * _W // _L

_LN2 = 0.6931471805599453


def _tc_body(thr_ref, score_ref, tgt_ref, cnt_lt_ref, sum_lt_ref, cnt_le_ref):
    i = pl.program_id(0)
    j = pl.program_id(1)

    @pl.when((i == 0) & (j == 0))
    def _init():
        cnt_lt_ref[0, 0] = 0.0
        sum_lt_ref[0, 0] = 0.0
        cnt_le_ref[0, 0] = 0.0

    x = score_ref[0]
    t = tgt_ref[0]
    m = jnp.max(x, axis=0)
    onehot = lax.broadcasted_iota(jnp.int32, x.shape, 0) == t[None]
    x_t = jnp.sum(jnp.where(onehot, x, 0.0), axis=0)
    e = jnp.exp(x - m[None])
    s = jnp.sum(e, axis=0)
    nll = jnp.log(s) + (m - x_t)
    lthr = thr_ref[0, 0]
    lt = -nll < lthr
    cnt_lt_ref[0, 0] += jnp.sum(lt.astype(jnp.float32))
    sum_lt_ref[0, 0] += jnp.sum(jnp.where(lt, nll, 0.0))
    cnt_le_ref[0, 0] += jnp.sum((-nll <= lthr).astype(jnp.float32))


def _tc_stats(score, target, thrv2):
    out = pl.pallas_call(
        _tc_body,
        grid=(_B_TC, 512 // _BH),
        in_specs=[
            pl.BlockSpec(memory_space=pltpu.SMEM),
            pl.BlockSpec((1, _C, _BH, _W), lambda i, j: (i, 0, j, 0)),
            pl.BlockSpec((1, _BH, _W), lambda i, j: (i, j, 0)),
        ],
        out_specs=[
            pl.BlockSpec(memory_space=pltpu.SMEM),
            pl.BlockSpec(memory_space=pltpu.SMEM),
            pl.BlockSpec(memory_space=pltpu.SMEM),
        ],
        out_shape=[jax.ShapeDtypeStruct((1, 1), jnp.float32)] * 3,
    )(thrv2, score, target)
    return out[0][0, 0], out[1][0, 0], out[2][0, 0]


def _vlog(s):
    bits = lax.bitcast_convert_type(s, jnp.int32)
    e = (bits >> 23) - 127
    mant = lax.bitcast_convert_type((bits & 0x7FFFFF) | 0x3F800000, jnp.float32)
    u = (mant - 1.0) / (mant + 1.0)
    u2 = u * u
    p = 1.0 + u2 * ((1.0 / 3.0) + u2 * ((1.0 / 5.0) + u2 * ((1.0 / 7.0) + u2 * (1.0 / 9.0))))
    return e.astype(jnp.float32) * _LN2 + 2.0 * u * p


def _sc_body(score_hbm, tgt_hbm, thr_hbm, out_hbm,
             buf_a, buf_b, tbuf_a, tbuf_b, thrv, accv,
             sem_a, sem_b, tsem_a, tsem_b):
    wid = lax.axis_index("s") * _NC + lax.axis_index("c")
    b = _B0 + wid // _QW
    base_row = (wid % _QW) * _RW

    pltpu.sync_copy(thr_hbm, thrv)
    thr = thrv[...]

    def start(j, buf, tbuf, sem, tsem):
        row = base_row + j * _CR
        for c in range(_C):
            pltpu.make_async_copy(score_hbm.at[b, c, pl.ds(row, _CR), :],
                                  buf.at[pl.ds(c * _CR, _CR), :], sem).start()
        pltpu.make_async_copy(tgt_hbm.at[b, pl.ds(row, _CR), :], tbuf, tsem).start()

    def wait(j, buf, tbuf, sem, tsem):
        row = base_row + j * _CR
        for c in range(_C):
            pltpu.make_async_copy(score_hbm.at[b, c, pl.ds(row, _CR), :],
                                  buf.at[pl.ds(c * _CR, _CR), :], sem).wait()
        pltpu.make_async_copy(tgt_hbm.at[b, pl.ds(row, _CR), :], tbuf, tsem).wait()

    start(0, buf_a, tbuf_a, sem_a, tsem_a)
    start(1, buf_b, tbuf_b, sem_b, tsem_b)

    one = jnp.ones((_L,), jnp.float32)
    zero = jnp.zeros((_L,), jnp.float32)

    def tree(op, vals):
        vals = list(vals)
        while len(vals) > 1:
            nxt = [op(vals[i], vals[i + 1]) for i in range(0, len(vals) - 1, 2)]
            if len(vals) % 2:
                nxt.append(vals[-1])
            vals = nxt
        return vals[0]

    def chunk(j, buf, tbuf, sem, tsem, accs):
        wait(j, buf, tbuf, sem, tsem)

        def one_group(r, wb, accs):
            c_lt, s_lt, c_le = accs
            xs = [buf[c * _CR + r, pl.ds(wb, _L)] for c in range(_C)]
            tv = tbuf[r, pl.ds(wb, _L)]
            x_t = tree(jnp.add,
                       [jnp.where(tv == c, xs[c], zero) for c in range(_C)])
            m = tree(jnp.maximum, xs)
            ssum = tree(jnp.add, [jnp.exp(x - m) for x in xs])
            nll = _vlog(ssum) + (m - x_t)
            lt = -nll < thr
            return (c_lt + jnp.where(lt, one, zero),
                    s_lt + jnp.where(lt, nll, zero),
                    c_le + jnp.where(-nll <= thr, one, zero))

        def grp(g, accs):
            g2 = g * 2
            r = g2 >> 5
            wb = (g2 & 31) * _L
            accs = one_group(r, wb, accs)
            return one_group(r, wb + _L, accs)

        accs = lax.fori_loop(0, _NG // 2, grp, accs)

        @pl.when(j + 2 < _NCHUNK)
        def _():
            start(j + 2, buf, tbuf, sem, tsem)

        return accs

    zero16 = jnp.zeros((_L,), jnp.float32)

    def pair(i, accs):
        accs = chunk(2 * i, buf_a, tbuf_a, sem_a, tsem_a, accs)
        accs = chunk(2 * i + 1, buf_b, tbuf_b, sem_b, tsem_b, accs)
        return accs

    accs = lax.fori_loop(0, _NCHUNK // 2, pair, (zero16, zero16, zero16))

    accv[pl.ds(0, _L)] = accs[0]
    accv[pl.ds(_L, _L)] = accs[1]
    accv[pl.ds(2 * _L, _L)] = accs[2]
    pltpu.sync_copy(accv, out_hbm.at[wid])


_sc_stats_kernel = functools.partial(
    pl.kernel,
    out_type=jax.ShapeDtypeStruct((_NW, 3 * _L), jnp.float32),
    mesh=plsc.VectorSubcoreMesh(core_axis_name="c", subcore_axis_name="s",
                                num_cores=_NC, num_subcores=_NS),
    compiler_params=pltpu.CompilerParams(use_tc_tiling_on_sc=True),
    scratch_types=[
        pltpu.VMEM((_C * _CR, _W), jnp.float32),
        pltpu.VMEM((_C * _CR, _W), jnp.float32),
        pltpu.VMEM((_CR, _W), jnp.int32),
        pltpu.VMEM((_CR, _W), jnp.int32),
        pltpu.VMEM((_L,), jnp.float32),
        pltpu.VMEM((3 * _L,), jnp.float32),
        pltpu.SemaphoreType.DMA,
        pltpu.SemaphoreType.DMA,
        pltpu.SemaphoreType.DMA,
        pltpu.SemaphoreType.DMA,
    ],
)(_sc_body)


def _stats(score, target, thr):
    lthr = jnp.log(jnp.asarray(thr, jnp.float32))
    sc_out = _sc_stats_kernel(score, target, jnp.full((_L,), lthr, jnp.float32))
    tc_lt, tc_sum, tc_le = _tc_stats(score, target, lthr.reshape(1, 1))
    return (tc_lt + jnp.sum(sc_out[:, 0:_L]),
            tc_sum + jnp.sum(sc_out[:, _L:2 * _L]),
            tc_le + jnp.sum(sc_out[:, 2 * _L:3 * _L]))


def kernel(score, target):
    kp1 = jnp.float32(_MIN_KEPT + 1)
    cnt_lt, sum_lt, cnt_le = _stats(score, target, _THRESH)

    def case_a(_):
        return sum_lt / jnp.maximum(cnt_lt, 1.0)

    def case_b(_):
        def cond(st):
            lo, hi = st
            return hi - lo > 1

        def body(st):
            lo, hi = st
            mid = (lo + hi) // 2
            t = lax.bitcast_convert_type(mid, jnp.float32)
            _, _, c_le = _stats(score, target, t)
            ge = c_le >= kp1
            return jnp.where(ge, lo, mid), jnp.where(ge, mid, hi)

        lo0 = jnp.int32(0x3F333333)
        hi0 = jnp.int32(0x3F800000)
        _, hi = lax.while_loop(cond, body, (lo0, hi0))
        vstar = lax.bitcast_convert_type(hi, jnp.float32)
        c_lt2, s_lt2, _ = _stats(score, target, vstar)
        return s_lt2 / jnp.maximum(c_lt2, 1.0)

    return lax.cond(cnt_le < kp1, case_b, case_a, None)


# speedup vs baseline: 2.9763x; 1.0001x over previous
import functools

import jax
import jax.numpy as jnp
from jax import lax
from jax.experimental import pallas as pl
from jax.experimental.pallas import tpu as pltpu
from jax.experimental.pallas import tpu_sc as plsc

_THRESH = 0.7
_MIN_KEPT = 100000
_C = 19
_B = 8
_B_TC = 6
_B0 = _B_TC
_BH = 128

_NC, _NS, _L = 2, 16, 16
_NW = _NC * _NS
_W = 512
_CR = 4
_PW = (_B - _B0) * 512 * _W // _NW
_RW = _PW // _W
_QW = 512 // _RW
_NCHUNK = _RW // _CR
_NG = _CR * _W // _L

_LN2 = 0.6931471805599453


def _tc_body(thr_ref, score_ref, tgt_ref, cnt_lt_ref, sum_lt_ref, cnt_le_ref):
    i = pl.program_id(0)
    j = pl.program_id(1)

    @pl.when((i == 0) & (j == 0))
    def _init():
        cnt_lt_ref[0, 0] = 0.0
        sum_lt_ref[0, 0] = 0.0
        cnt_le_ref[0, 0] = 0.0

    x = score_ref[0]
    t = tgt_ref[0]
    m = jnp.max(x, axis=0)
    onehot = lax.broadcasted_iota(jnp.int32, x.shape, 0) == t[None]
    x_t = jnp.sum(jnp.where(onehot, x, 0.0), axis=0)
    e = jnp.exp(x - m[None])
    s = jnp.sum(e, axis=0)
    nll = jnp.log(s) + (m - x_t)
    lthr = thr_ref[0, 0]
    lt = -nll < lthr
    cnt_lt_ref[0, 0] += jnp.sum(lt.astype(jnp.float32))
    sum_lt_ref[0, 0] += jnp.sum(jnp.where(lt, nll, 0.0))
    cnt_le_ref[0, 0] += jnp.sum((-nll <= lthr).astype(jnp.float32))


def _tc_stats(score, target, thrv2):
    out = pl.pallas_call(
        _tc_body,
        grid=(_B_TC, 512 // _BH),
        in_specs=[
            pl.BlockSpec(memory_space=pltpu.SMEM),
            pl.BlockSpec((1, _C, _BH, _W), lambda i, j: (i, 0, j, 0)),
            pl.BlockSpec((1, _BH, _W), lambda i, j: (i, j, 0)),
        ],
        out_specs=[
            pl.BlockSpec(memory_space=pltpu.SMEM),
            pl.BlockSpec(memory_space=pltpu.SMEM),
            pl.BlockSpec(memory_space=pltpu.SMEM),
        ],
        out_shape=[jax.ShapeDtypeStruct((1, 1), jnp.float32)] * 3,
    )(thrv2, score, target)
    return out[0][0, 0], out[1][0, 0], out[2][0, 0]


def _vlog(s):
    bits = lax.bitcast_convert_type(s, jnp.int32)
    e = (bits >> 23) - 127
    mant = lax.bitcast_convert_type((bits & 0x7FFFFF) | 0x3F800000, jnp.float32)
    u = (mant - 1.0) / (mant + 1.0)
    u2 = u * u
    p = 1.0 + u2 * ((1.0 / 3.0) + u2 * ((1.0 / 5.0) + u2 * ((1.0 / 7.0) + u2 * (1.0 / 9.0))))
    return e.astype(jnp.float32) * _LN2 + 2.0 * u * p


def _sc_body(score_hbm, tgt_hbm, thr_hbm, out_hbm,
             buf_a, buf_b, tbuf_a, tbuf_b, thrv, accv,
             sem_a, sem_b, tsem_a, tsem_b):
    wid = lax.axis_index("s") * _NC + lax.axis_index("c")
    b = _B0 + wid // _QW
    base_row = (wid % _QW) * _RW

    pltpu.sync_copy(thr_hbm, thrv)
    thr = thrv[...]

    def start(j, buf, tbuf, sem, tsem):
        row = base_row + j * _CR
        for c in range(_C):
            pltpu.make_async_copy(score_hbm.at[b, c, pl.ds(row, _CR), :],
                                  buf.at[pl.ds(c * _CR, _CR), :], sem).start()
        pltpu.make_async_copy(tgt_hbm.at[b, pl.ds(row, _CR), :], tbuf, tsem).start()

    def wait(j, buf, tbuf, sem, tsem):
        row = base_row + j * _CR
        for c in range(_C):
            pltpu.make_async_copy(score_hbm.at[b, c, pl.ds(row, _CR), :],
                                  buf.at[pl.ds(c * _CR, _CR), :], sem).wait()
        pltpu.make_async_copy(tgt_hbm.at[b, pl.ds(row, _CR), :], tbuf, tsem).wait()

    start(0, buf_a, tbuf_a, sem_a, tsem_a)
    start(1, buf_b, tbuf_b, sem_b, tsem_b)

    one = jnp.ones((_L,), jnp.float32)
    zero = jnp.zeros((_L,), jnp.float32)

    def tree(op, vals):
        vals = list(vals)
        while len(vals) > 1:
            nxt = [op(vals[i], vals[i + 1]) for i in range(0, len(vals) - 1, 2)]
            if len(vals) % 2:
                nxt.append(vals[-1])
            vals = nxt
        return vals[0]

    def chunk(j, buf, tbuf, sem, tsem, accs):
        wait(j, buf, tbuf, sem, tsem)

        def one_group(r, wb, accs):
            c_lt, s_lt, c_le = accs
            xs = [buf[c * _CR + r, pl.ds(wb, _L)] for c in range(_C)]
            tv = tbuf[r, pl.ds(wb, _L)]
            x_t = tree(jnp.add,
                       [jnp.where(tv == c, xs[c], zero) for c in range(_C)])
            m = tree(jnp.maximum, xs)
            ssum = tree(jnp.add, [jnp.exp(x - m) for x in xs])
            nll = _vlog(ssum) + (m - x_t)
            lt = -nll < thr
            return (c_lt + jnp.where(lt, one, zero),
                    s_lt + jnp.where(lt, nll, zero),
                    c_le + jnp.where(-nll <= thr, one, zero))

        def grp(g, accs):
            g2 = g * 2
            r = g2 >> 5
            wb = (g2 & 31) * _L
            accs = one_group(r, wb, accs)
            return one_group(r, wb + _L, accs)

        accs = lax.fori_loop(0, _NG // 2, grp, accs)

        @pl.when(j + 2 < _NCHUNK)
        def _():
            start(j + 2, buf, tbuf, sem, tsem)

        return accs

    zero16 = jnp.zeros((_L,), jnp.float32)

    def pair(i, accs):
        accs = chunk(2 * i, buf_a, tbuf_a, sem_a, tsem_a, accs)
        accs = chunk(2 * i + 1, buf_b, tbuf_b, sem_b, tsem_b, accs)
        return accs

    accs = lax.fori_loop(0, _NCHUNK // 2, pair, (zero16, zero16, zero16))

    accv[pl.ds(0, _L)] = accs[0]
    accv[pl.ds(_L, _L)] = accs[1]
    accv[pl.ds(2 * _L, _L)] = accs[2]
    pltpu.sync_copy(accv, out_hbm.at[wid])


@functools.cache
def _sc_stats_kernel():
    return functools.partial(
        pl.kernel,
        out_type=jax.ShapeDtypeStruct((_NW, 3 * _L), jnp.float32),
        mesh=plsc.VectorSubcoreMesh(core_axis_name="c", subcore_axis_name="s",
                                    num_cores=_NC, num_subcores=_NS),
        compiler_params=pltpu.CompilerParams(use_tc_tiling_on_sc=True),
        scratch_types=[
            pltpu.VMEM((_C * _CR, _W), jnp.float32),
            pltpu.VMEM((_C * _CR, _W), jnp.float32),
            pltpu.VMEM((_CR, _W), jnp.int32),
            pltpu.VMEM((_CR, _W), jnp.int32),
            pltpu.VMEM((_L,), jnp.float32),
            pltpu.VMEM((3 * _L,), jnp.float32),
            pltpu.SemaphoreType.DMA,
            pltpu.SemaphoreType.DMA,
            pltpu.SemaphoreType.DMA,
            pltpu.SemaphoreType.DMA,
        ],
    )(_sc_body)


def _stats(score, target, thr):
    lthr = jnp.log(jnp.asarray(thr, jnp.float32))
    sc_out = _sc_stats_kernel()(score, target, jnp.full((_L,), lthr, jnp.float32))
    tc_lt, tc_sum, tc_le = _tc_stats(score, target, lthr.reshape(1, 1))
    return (tc_lt + jnp.sum(sc_out[:, 0:_L]),
            tc_sum + jnp.sum(sc_out[:, _L:2 * _L]),
            tc_le + jnp.sum(sc_out[:, 2 * _L:3 * _L]))


def kernel(score, target):
    kp1 = jnp.float32(_MIN_KEPT + 1)
    cnt_lt, sum_lt, cnt_le = _stats(score, target, _THRESH)

    def case_a(_):
        return sum_lt / jnp.maximum(cnt_lt, 1.0)

    def case_b(_):
        def cond(st):
            lo, hi = st
            return hi - lo > 1

        def body(st):
            lo, hi = st
            mid = (lo + hi) // 2
            t = lax.bitcast_convert_type(mid, jnp.float32)
            _, _, c_le = _stats(score, target, t)
            ge = c_le >= kp1
            return jnp.where(ge, lo, mid), jnp.where(ge, mid, hi)

        lo0 = jnp.int32(0x3F333333)
        hi0 = jnp.int32(0x3F800000)
        _, hi = lax.while_loop(cond, body, (lo0, hi0))
        vstar = lax.bitcast_convert_type(hi, jnp.float32)
        c_lt2, s_lt2, _ = _stats(score, target, vstar)
        return s_lt2 / jnp.maximum(c_lt2, 1.0)

    return lax.cond(cnt_le < kp1, case_b, case_a, None)
